# baseline (speedup 1.0000x reference)
import jax
import jax.numpy as jnp
from jax.experimental import pallas as pl
from jax.experimental.pallas import tpu as pltpu

_MAJ = 100
_BMAJ = 10


def _fused_kernel(p_ref, q_ref, ic_ref, wcat_ref, user_out_ref, item_out_ref):
    f32 = jnp.float32
    w = wcat_ref[0:128, :]
    weu = wcat_ref[128:256, :]
    wei_top = wcat_ref[256:320, :]
    wei_bot = wcat_ref[320:384, :]
    p = p_ref[...].reshape(_BMAJ * 1000, 128)
    q = q_ref[...]
    ic = ic_ref[...].reshape(_BMAJ * 1000, 128)
    user = jnp.dot(p, weu, preferred_element_type=f32)
    w_fold = jnp.dot(w, wei_bot, preferred_element_type=f32)
    item = (jnp.dot(q, wei_top, preferred_element_type=f32)
            + jnp.dot(ic, w_fold, preferred_element_type=f32))
    user_out_ref[...] = user.reshape(_BMAJ, 1000, 64)
    item_out_ref[...] = item.reshape(_BMAJ, 1000, 64)


@jax.jit
def kernel(P, Q, item_content, W, weu, wei):
    n = P.shape[0]
    d = weu.shape[1]
    wcat = jnp.concatenate([W, weu, wei], axis=0)
    P3 = P.reshape(_MAJ, 1000, P.shape[1])
    ic3 = item_content.reshape(_MAJ, 1000, item_content.shape[1])
    grid = (_MAJ // _BMAJ,)
    row_block = lambda i: (i, 0, 0)
    user3, item3 = pl.pallas_call(
        _fused_kernel,
        grid=grid,
        in_specs=[
            pl.BlockSpec((_BMAJ, 1000, P.shape[1]), row_block),
            pl.BlockSpec((_BMAJ * 1000, Q.shape[1]), lambda i: (i, 0)),
            pl.BlockSpec((_BMAJ, 1000, item_content.shape[1]), row_block),
            pl.BlockSpec(wcat.shape, lambda i: (0, 0)),
        ],
        out_specs=[
            pl.BlockSpec((_BMAJ, 1000, d), row_block),
            pl.BlockSpec((_BMAJ, 1000, d), row_block),
        ],
        out_shape=[
            jax.ShapeDtypeStruct((_MAJ, 1000, d), jnp.float32),
            jax.ShapeDtypeStruct((_MAJ, 1000, d), jnp.float32),
        ],
        compiler_params=pltpu.CompilerParams(
            dimension_semantics=("parallel",),
        ),
    )(P3, Q, ic3, wcat)
    return (user3.reshape(n, d), item3.reshape(n, d))
